# Initial kernel scaffold; baseline (speedup 1.0000x reference)
import jax
import jax.numpy as jnp
from jax.experimental import pallas as pl


def kernel(x, pos, batch, channel_weights, edge_index, edge_weights, lig_flag, chains, emb, gnn_w1, gnn_b1, gnn_w2, gnn_b2, gnn_lin_in, gnn_lin_out, ln_gamma, ln_beta, post_w, aff_w1, aff_b1, aff_w2, aff_b2, prop_w1, prop_b1, prop_w2, prop_b2):
    raise NotImplementedError("write your pallas kernel here")


# speedup vs baseline: 2.2871x; 2.2871x over previous
import functools

import jax
import jax.numpy as jnp
from jax import lax
from jax.experimental import pallas as pl
from jax.experimental.pallas import tpu as pltpu
from jax.experimental.pallas import tpu_sc as plsc

_R_CUTOFF = 4.5
_GAMMA = 10.0
_LN2 = 0.6931471805599453

_NC = 2
_NS = 16
_CH = 128


def _pick_blk(total, cands):
    return next(c for c in cands if total % c == 0)


def _ssp(v):
    return jnp.maximum(v, 0.0) + jnp.log(1.0 + jnp.exp(-jnp.abs(v))) - _LN2


def _silu(v):
    return v / (1.0 + jnp.exp(-v))


def _embed_body(x_ref, emb_ref, lin_ref, s_ref, slin_ref):
    xv = x_ref[...]
    ids = lax.broadcasted_iota(jnp.int32, (1, 128), 1)
    onehot = (xv == ids).astype(jnp.float32)
    s = jnp.dot(onehot, emb_ref[...], preferred_element_type=jnp.float32)
    s_ref[...] = s
    slin_ref[...] = jnp.dot(s, lin_ref[...], preferred_element_type=jnp.float32)


def _embed(x2d, emb_pad, lin_in0, n, blk):
    grid = n // blk
    return pl.pallas_call(
        _embed_body,
        grid=(grid,),
        in_specs=[
            pl.BlockSpec((blk, 1), lambda i: (i, 0)),
            pl.BlockSpec((128, 128), lambda i: (0, 0)),
            pl.BlockSpec((128, 128), lambda i: (0, 0)),
        ],
        out_specs=[
            pl.BlockSpec((blk, 128), lambda i: (i, 0)),
            pl.BlockSpec((blk, 128), lambda i: (i, 0)),
        ],
        out_shape=[
            jax.ShapeDtypeStruct((n, 128), jnp.float32),
            jax.ShapeDtypeStruct((n, 128), jnp.float32),
        ],
    )(x2d, emb_pad, lin_in0)


def _filt_body(ew_ref, w1_ref, b1_ref, w2_ref, b2_ref, out_ref, *, num_radial):
    ew = ew_ref[...]
    mu = lax.broadcasted_iota(jnp.int32, (1, num_radial), 1).astype(
        jnp.float32) * (_R_CUTOFF / (num_radial - 1))
    rbf = jnp.exp(-_GAMMA * (ew - mu) ** 2)
    h = _ssp(jnp.dot(rbf, w1_ref[0], preferred_element_type=jnp.float32)
             + b1_ref[0])
    out_ref[0] = (jnp.dot(h, w2_ref[0], preferred_element_type=jnp.float32)
                  + b2_ref[0])


def _filters(ew2d, w1, b1, w2, b2, depth, e, num_radial, blk):
    grid = (depth, e // blk)
    return pl.pallas_call(
        functools.partial(_filt_body, num_radial=num_radial),
        grid=grid,
        in_specs=[
            pl.BlockSpec((blk, 1), lambda l, i: (i, 0)),
            pl.BlockSpec((1, num_radial, 128), lambda l, i: (l, 0, 0)),
            pl.BlockSpec((1, 1, 128), lambda l, i: (l, 0, 0)),
            pl.BlockSpec((1, 128, 128), lambda l, i: (l, 0, 0)),
            pl.BlockSpec((1, 1, 128), lambda l, i: (l, 0, 0)),
        ],
        out_specs=pl.BlockSpec((1, blk, 128), lambda l, i: (l, i, 0)),
        out_shape=jax.ShapeDtypeStruct((depth, e, 128), jnp.float32),
    )(ew2d, w1, b1.reshape(depth, 1, 128), w2, b2.reshape(depth, 1, 128))


def _tile_copy(src, dst, sid, n):
    npt = (n // _NS) // 8 * 8
    rem = n - _NS * npt
    off = pl.multiple_of(sid * npt, 8)
    pltpu.sync_copy(src.at[pl.ds(off, npt)], dst.at[pl.ds(off, npt)])
    if rem:
        @pl.when(sid == _NS - 1)
        def _():
            pltpu.sync_copy(src.at[pl.ds(_NS * npt, rem)],
                            dst.at[pl.ds(_NS * npt, rem)])


def _deg_body(row2d, zeros_ns, out, rowv, onesb, degsh, *, n, rows_per_core,
              rows_per_tile):
    cid = lax.axis_index("c")
    sid = lax.axis_index("s")
    _tile_copy(zeros_ns, degsh, sid, n)

    def fill_row(r, _):
        for c8 in range(8):
            onesb[r, pl.ds(c8 * 16, 16)] = jnp.full((16,), 1.0, jnp.float32)
        return 0

    lax.fori_loop(0, _CH, fill_row, 0)
    plsc.subcore_barrier()

    base = cid * rows_per_core + sid * rows_per_tile

    def do_chunk(ck):
        pltpu.sync_copy(row2d.at[ck], rowv)
        pltpu.sync_copy(onesb, degsh.at[rowv], add=True)

    def chunk_loop(i, _):
        do_chunk(base + i)
        return 0

    lax.fori_loop(0, rows_per_tile, chunk_loop, 0)
    nextra = rows_per_core - _NS * rows_per_tile

    @pl.when(sid < nextra)
    def _():
        do_chunk(cid * rows_per_core + _NS * rows_per_tile + sid)

    plsc.subcore_barrier()
    _tile_copy(degsh, out.at[cid], sid, n)


def _degree(row2d, zeros_ns, n, e):
    nrows = e // _CH
    rows_per_core = nrows // _NC
    rows_per_tile = rows_per_core // _NS
    mesh = plsc.VectorSubcoreMesh(core_axis_name="c", subcore_axis_name="s",
                                  num_cores=_NC, num_subcores=_NS)
    body = functools.partial(_deg_body, n=n, rows_per_core=rows_per_core,
                             rows_per_tile=rows_per_tile)
    f = pl.kernel(
        body,
        out_type=jax.ShapeDtypeStruct((_NC, n, 128), jnp.float32),
        mesh=mesh,
        scratch_types=[
            pltpu.VMEM((_CH,), jnp.int32),
            pltpu.VMEM((_CH, 128), jnp.float32),
            pltpu.VMEM_SHARED((n, 128), jnp.float32),
        ],
    )
    return f(row2d, zeros_ns)


def _cfconv_body(slin, filt, row2d, col2d, zeros_ns, out, colv, rowv, xr, fc,
                 aggsh, sem, *, n, rows_per_core, rows_per_tile):
    cid = lax.axis_index("c")
    sid = lax.axis_index("s")
    _tile_copy(zeros_ns, aggsh, sid, n)
    plsc.subcore_barrier()

    base = cid * rows_per_core + sid * rows_per_tile

    def do_chunk(ck):
        pltpu.sync_copy(col2d.at[ck], colv)
        pltpu.sync_copy(row2d.at[ck], rowv)
        gather = pltpu.async_copy(slin.at[colv], xr, sem)
        off = pl.multiple_of(ck * _CH, _CH)
        pltpu.sync_copy(filt.at[pl.ds(off, _CH)], fc)
        gather.wait()

        def mul_row(r, _):
            for c8 in range(8):
                sl = pl.ds(c8 * 16, 16)
                xr[r, sl] = xr[r, sl] * fc[r, sl]
            return 0

        lax.fori_loop(0, _CH, mul_row, 0)
        pltpu.sync_copy(xr, aggsh.at[rowv], add=True)

    def chunk_loop(i, _):
        do_chunk(base + i)
        return 0

    lax.fori_loop(0, rows_per_tile, chunk_loop, 0)
    nextra = rows_per_core - _NS * rows_per_tile

    @pl.when(sid < nextra)
    def _():
        do_chunk(cid * rows_per_core + _NS * rows_per_tile + sid)

    plsc.subcore_barrier()
    _tile_copy(aggsh, out.at[cid], sid, n)


def _cfconv(slin, filt_l, row2d, col2d, zeros_ns, n, e):
    nrows = e // _CH
    rows_per_core = nrows // _NC
    rows_per_tile = rows_per_core // _NS
    mesh = plsc.VectorSubcoreMesh(core_axis_name="c", subcore_axis_name="s",
                                  num_cores=_NC, num_subcores=_NS)
    body = functools.partial(_cfconv_body, n=n, rows_per_core=rows_per_core,
                             rows_per_tile=rows_per_tile)
    f = pl.kernel(
        body,
        out_type=jax.ShapeDtypeStruct((_NC, n, 128), jnp.float32),
        mesh=mesh,
        scratch_types=[
            pltpu.VMEM((_CH,), jnp.int32),
            pltpu.VMEM((_CH,), jnp.int32),
            pltpu.VMEM((_CH, 128), jnp.float32),
            pltpu.VMEM((_CH, 128), jnp.float32),
            pltpu.VMEM_SHARED((n, 128), jnp.float32),
            pltpu.SemaphoreType.DMA,
        ],
    )
    return f(slin, filt_l, row2d, col2d, zeros_ns)


def _update_body(s_ref, agg_ref, deg_ref, lo_ref, ln_ref, s_out, slin_out, *,
                 has_next):
    agg = agg_ref[0] + agg_ref[1]
    deg = jnp.maximum(deg_ref[0, :, 0:1] + deg_ref[1, :, 0:1], 1.0)
    u = _ssp(agg / deg)
    s_new = s_ref[...] + jnp.dot(u, lo_ref[...],
                                 preferred_element_type=jnp.float32)
    s_out[...] = s_new
    if has_next:
        slin_out[...] = jnp.dot(s_new, ln_ref[...],
                                preferred_element_type=jnp.float32)
    else:
        slin_out[...] = s_new


def _update(s, aggp, degp, lin_out_l, lin_in_next, n, blk, has_next):
    grid = n // blk
    return pl.pallas_call(
        functools.partial(_update_body, has_next=has_next),
        grid=(grid,),
        in_specs=[
            pl.BlockSpec((blk, 128), lambda i: (i, 0)),
            pl.BlockSpec((_NC, blk, 128), lambda i: (0, i, 0)),
            pl.BlockSpec((_NC, blk, 128), lambda i: (0, i, 0)),
            pl.BlockSpec((128, 128), lambda i: (0, 0)),
            pl.BlockSpec((128, 128), lambda i: (0, 0)),
        ],
        out_specs=[
            pl.BlockSpec((blk, 128), lambda i: (i, 0)),
            pl.BlockSpec((blk, 128), lambda i: (i, 0)),
        ],
        out_shape=[
            jax.ShapeDtypeStruct((n, 128), jnp.float32),
            jax.ShapeDtypeStruct((n, 128), jnp.float32),
        ],
    )(s, aggp, degp, lin_out_l, lin_in_next)


def _final_body(s_ref, batch_ref, lig_ref, ch_ref, g_ref, be_ref, pw_ref,
                aw1_ref, ab1_ref, aw2_ref, ab2_ref, pw1_ref, pb1_ref, pw2_ref,
                pb2_ref, aff_ref, prop_ref, *, n, n_graphs, n_chains, n_aff,
                n_prop, prop_classes):
    s = s_ref[...]
    m = jnp.mean(s, axis=1, keepdims=True)
    d = s - m
    var = jnp.mean(d * d, axis=1, keepdims=True)
    sn = g_ref[...] * d * lax.rsqrt(var + 1e-5) + be_ref[...]
    sp = jnp.dot(sn, pw_ref[...], preferred_element_type=jnp.float32)

    bt = batch_ref[...]
    gids = lax.broadcasted_iota(jnp.int32, (n_graphs, 1), 0)
    oh_g = (bt == gids).astype(jnp.float32)
    gsum = jnp.dot(oh_g, sp, preferred_element_type=jnp.float32)
    gcnt = jnp.maximum(jnp.sum(oh_g, axis=1, keepdims=True), 1.0)
    g = gsum / gcnt

    for i in range(n_aff):
        h = _silu(jnp.dot(g, aw1_ref[i], preferred_element_type=jnp.float32)
                  + ab1_ref[i])
        val = jnp.sum(h * aw2_ref[i], axis=1, keepdims=True) + ab2_ref[i, 0, 0]
        aff_ref[i] = jnp.broadcast_to(val, (n_graphs, 128))

    w = (lig_ref[...] != 0).astype(jnp.float32)
    cidx = ch_ref[...] - 1
    cids = lax.broadcasted_iota(jnp.int32, (n_chains, 1), 0)
    oh_c = (cidx == cids).astype(jnp.float32) * w
    csum = jnp.dot(oh_c, sp, preferred_element_type=jnp.float32)
    ccnt = jnp.maximum(jnp.sum(oh_c, axis=1, keepdims=True), 1.0)
    cg = csum / ccnt

    for i in range(n_prop):
        h = _silu(jnp.dot(cg, pw1_ref[i], preferred_element_type=jnp.float32)
                  + pb1_ref[i])
        for c in range(prop_classes):
            val = (jnp.sum(h * pw2_ref[i, c][None, :], axis=1, keepdims=True)
                   + pb2_ref[i, 0, c])
            prop_ref[i * prop_classes + c] = jnp.broadcast_to(
                val, (n_chains, 128))


def _final(s, batch_r, lig_r, ch_r, ln_gamma, ln_beta, post_w, aff_w1, aff_b1,
           aff_w2, aff_b2, prop_w1, prop_b1, prop_w2, prop_b2, n, n_graphs,
           n_chains):
    n_aff = aff_w1.shape[0]
    n_prop = prop_w1.shape[0]
    prop_classes = prop_w2.shape[2]
    body = functools.partial(
        _final_body, n=n, n_graphs=n_graphs, n_chains=n_chains, n_aff=n_aff,
        n_prop=n_prop, prop_classes=prop_classes)
    return pl.pallas_call(
        body,
        out_shape=[
            jax.ShapeDtypeStruct((n_aff, n_graphs, 128), jnp.float32),
            jax.ShapeDtypeStruct((n_prop * prop_classes, n_chains, 128),
                                 jnp.float32),
        ],
    )(s, batch_r, lig_r, ch_r, ln_gamma.reshape(1, 128),
      ln_beta.reshape(1, 128), post_w, aff_w1,
      aff_b1.reshape(n_aff, 1, 128), aff_w2.reshape(n_aff, 1, 128),
      aff_b2.reshape(n_aff, 1, 1), prop_w1, prop_b1.reshape(n_prop, 1, 128),
      jnp.transpose(prop_w2, (0, 2, 1)),
      prop_b2.reshape(n_prop, 1, prop_classes))


def kernel(x, pos, batch, channel_weights, edge_index, edge_weights, lig_flag,
           chains, emb, gnn_w1, gnn_b1, gnn_w2, gnn_b2, gnn_lin_in,
           gnn_lin_out, ln_gamma, ln_beta, post_w, aff_w1, aff_b1, aff_w2,
           aff_b2, prop_w1, prop_b1, prop_w2, prop_b2):
    n = x.shape[0]
    e = edge_index.shape[1]
    depth = gnn_w1.shape[0]
    num_radial = gnn_w1.shape[1]
    n_graphs = 16
    n_chains = 8
    blk_n = _pick_blk(n, (2000, 1000, 500, 250, 200, 8))
    blk_e = _pick_blk(e, (2000, 2048, 1024, 512, 256, 128))

    x2d = x.astype(jnp.int32).reshape(n, 1)
    row2d = edge_index[0].astype(jnp.int32).reshape(e // _CH, _CH)
    col2d = edge_index[1].astype(jnp.int32).reshape(e // _CH, _CH)
    ew2d = edge_weights.reshape(e, 1)
    emb_pad = jnp.zeros((128, 128), jnp.float32).at[:emb.shape[0]].set(emb)
    zeros_ns = jnp.zeros((n, 128), jnp.float32)

    s, slin = _embed(x2d, emb_pad, gnn_lin_in[0], n, blk_n)
    filt = _filters(ew2d, gnn_w1, gnn_b1, gnn_w2, gnn_b2, depth, e,
                    num_radial, blk_e)
    degp = _degree(row2d, zeros_ns, n, e)

    for l in range(depth):
        aggp = _cfconv(slin, filt[l], row2d, col2d, zeros_ns, n, e)
        has_next = l + 1 < depth
        nxt = gnn_lin_in[l + 1] if has_next else gnn_lin_in[0]
        s, slin = _update(s, aggp, degp, gnn_lin_out[l], nxt, n, blk_n,
                          has_next)

    aff_pad, prop_pad = _final(
        s, batch.astype(jnp.int32).reshape(1, n),
        lig_flag.astype(jnp.int32).reshape(1, n),
        chains.astype(jnp.int32).reshape(1, n), ln_gamma, ln_beta, post_w,
        aff_w1, aff_b1, aff_w2, aff_b2, prop_w1, prop_b1, prop_w2, prop_b2,
        n, n_graphs, n_chains)

    n_aff = aff_w1.shape[0]
    n_prop = prop_w1.shape[0]
    prop_classes = prop_w2.shape[2]
    aff_out = aff_pad[:, :, :1]
    prop_out = jnp.transpose(
        prop_pad[:, :, 0].reshape(n_prop, prop_classes, n_chains), (0, 2, 1))
    return aff_out, prop_out


# speedup vs baseline: 3.1842x; 1.3922x over previous
import functools

import jax
import jax.numpy as jnp
from jax import lax
from jax.experimental import pallas as pl
from jax.experimental.pallas import tpu as pltpu
from jax.experimental.pallas import tpu_sc as plsc

_R_CUTOFF = 4.5
_GAMMA = 10.0
_LN2 = 0.6931471805599453

_NC = 2
_NS = 16
_CH = 128


def _pick_blk(total, cands):
    return next(c for c in cands if total % c == 0)


def _ssp(v):
    return jnp.maximum(v, 0.0) + jnp.log(1.0 + jnp.exp(-jnp.abs(v))) - _LN2


def _silu(v):
    return v / (1.0 + jnp.exp(-v))


def _embed_body(x_ref, emb_ref, lin_ref, s_ref, slin_ref):
    xv = x_ref[...]
    ids = lax.broadcasted_iota(jnp.int32, (1, 128), 1)
    onehot = (xv == ids).astype(jnp.float32)
    s = jnp.dot(onehot, emb_ref[...], preferred_element_type=jnp.float32)
    s_ref[...] = s
    slin_ref[...] = jnp.dot(s, lin_ref[...], preferred_element_type=jnp.float32)


def _embed(x2d, emb_pad, lin_in0, n, blk):
    grid = n // blk
    return pl.pallas_call(
        _embed_body,
        grid=(grid,),
        in_specs=[
            pl.BlockSpec((blk, 1), lambda i: (i, 0)),
            pl.BlockSpec((128, 128), lambda i: (0, 0)),
            pl.BlockSpec((128, 128), lambda i: (0, 0)),
        ],
        out_specs=[
            pl.BlockSpec((blk, 128), lambda i: (i, 0)),
            pl.BlockSpec((blk, 128), lambda i: (i, 0)),
        ],
        out_shape=[
            jax.ShapeDtypeStruct((n, 128), jnp.float32),
            jax.ShapeDtypeStruct((n, 128), jnp.float32),
        ],
    )(x2d, emb_pad, lin_in0)


def _filt_body(ew_ref, w1_ref, b1_ref, w2_ref, b2_ref, out_ref, *, num_radial):
    ew = ew_ref[...]
    mu = lax.broadcasted_iota(jnp.int32, (1, num_radial), 1).astype(
        jnp.float32) * (_R_CUTOFF / (num_radial - 1))
    rbf = jnp.exp(-_GAMMA * (ew - mu) ** 2)
    h = _ssp(jnp.dot(rbf, w1_ref[...], preferred_element_type=jnp.float32)
             + b1_ref[...])
    out_ref[...] = (jnp.dot(h, w2_ref[...], preferred_element_type=jnp.float32)
                    + b2_ref[...])


def _filters1(ew2d, w1l, b1l, w2l, b2l, e, num_radial, blk):
    return pl.pallas_call(
        functools.partial(_filt_body, num_radial=num_radial),
        grid=(e // blk,),
        in_specs=[
            pl.BlockSpec((blk, 1), lambda i: (i, 0)),
            pl.BlockSpec((num_radial, 128), lambda i: (0, 0)),
            pl.BlockSpec((1, 128), lambda i: (0, 0)),
            pl.BlockSpec((128, 128), lambda i: (0, 0)),
            pl.BlockSpec((1, 128), lambda i: (0, 0)),
        ],
        out_specs=pl.BlockSpec((blk, 128), lambda i: (i, 0)),
        out_shape=jax.ShapeDtypeStruct((e, 128), jnp.float32),
    )(ew2d, w1l, b1l.reshape(1, 128), w2l, b2l.reshape(1, 128))


def _tile_copy(src, dst, sid, n):
    npt = (n // _NS) // 8 * 8
    rem = n - _NS * npt
    off = pl.multiple_of(sid * npt, 8)
    pltpu.sync_copy(src.at[pl.ds(off, npt)], dst.at[pl.ds(off, npt)])
    if rem:
        @pl.when(sid == _NS - 1)
        def _():
            pltpu.sync_copy(src.at[pl.ds(_NS * npt, rem)],
                            dst.at[pl.ds(_NS * npt, rem)])


def _deg_body(row2d, zeros_ns, out, rowv, onesb, degsh, *, n, rows_per_core,
              rows_per_tile):
    cid = lax.axis_index("c")
    sid = lax.axis_index("s")
    _tile_copy(zeros_ns, degsh, sid, n)

    def fill_row(r, _):
        for c8 in range(8):
            onesb[r, pl.ds(c8 * 16, 16)] = jnp.full((16,), 1.0, jnp.float32)
        return 0

    lax.fori_loop(0, _CH, fill_row, 0)
    plsc.subcore_barrier()

    base = cid * rows_per_core + sid * rows_per_tile

    def do_chunk(ck):
        pltpu.sync_copy(row2d.at[ck], rowv)
        pltpu.sync_copy(onesb, degsh.at[rowv], add=True)

    def chunk_loop(i, _):
        do_chunk(base + i)
        return 0

    lax.fori_loop(0, rows_per_tile, chunk_loop, 0)
    nextra = rows_per_core - _NS * rows_per_tile

    @pl.when(sid < nextra)
    def _():
        do_chunk(cid * rows_per_core + _NS * rows_per_tile + sid)

    plsc.subcore_barrier()
    _tile_copy(degsh, out.at[cid], sid, n)


def _degree(row2d, zeros_ns, n, e):
    nrows = e // _CH
    rows_per_core = nrows // _NC
    rows_per_tile = rows_per_core // _NS
    mesh = plsc.VectorSubcoreMesh(core_axis_name="c", subcore_axis_name="s",
                                  num_cores=_NC, num_subcores=_NS)
    body = functools.partial(_deg_body, n=n, rows_per_core=rows_per_core,
                             rows_per_tile=rows_per_tile)
    f = pl.kernel(
        body,
        out_type=jax.ShapeDtypeStruct((_NC, n, 128), jnp.float32),
        mesh=mesh,
        scratch_types=[
            pltpu.VMEM((_CH,), jnp.int32),
            pltpu.VMEM((_CH, 128), jnp.float32),
            pltpu.VMEM_SHARED((n, 128), jnp.float32),
        ],
    )
    return f(row2d, zeros_ns)


def _cfconv_body(slin, filt, row2d, col2d, zeros_ns, out, colv, rowv, xr, fc,
                 aggsh, sem, *, n, rows_per_core, rows_per_tile):
    cid = lax.axis_index("c")
    sid = lax.axis_index("s")
    _tile_copy(zeros_ns, aggsh, sid, n)
    plsc.subcore_barrier()

    base = cid * rows_per_core + sid * rows_per_tile

    def do_chunk(ck):
        pltpu.sync_copy(col2d.at[ck], colv)
        pltpu.sync_copy(row2d.at[ck], rowv)
        gather = pltpu.async_copy(slin.at[colv], xr, sem)
        off = pl.multiple_of(ck * _CH, _CH)
        pltpu.sync_copy(filt.at[pl.ds(off, _CH)], fc)
        gather.wait()

        def mul_row(r, _):
            for c8 in range(8):
                sl = pl.ds(c8 * 16, 16)
                xr[r, sl] = xr[r, sl] * fc[r, sl]
            return 0

        lax.fori_loop(0, _CH, mul_row, 0)
        pltpu.sync_copy(xr, aggsh.at[rowv], add=True)

    def chunk_loop(i, _):
        do_chunk(base + i)
        return 0

    lax.fori_loop(0, rows_per_tile, chunk_loop, 0)
    nextra = rows_per_core - _NS * rows_per_tile

    @pl.when(sid < nextra)
    def _():
        do_chunk(cid * rows_per_core + _NS * rows_per_tile + sid)

    plsc.subcore_barrier()
    _tile_copy(aggsh, out.at[cid], sid, n)


def _cfconv(slin, filt_l, row2d, col2d, zeros_ns, n, e):
    nrows = e // _CH
    rows_per_core = nrows // _NC
    rows_per_tile = rows_per_core // _NS
    mesh = plsc.VectorSubcoreMesh(core_axis_name="c", subcore_axis_name="s",
                                  num_cores=_NC, num_subcores=_NS)
    body = functools.partial(_cfconv_body, n=n, rows_per_core=rows_per_core,
                             rows_per_tile=rows_per_tile)
    f = pl.kernel(
        body,
        out_type=jax.ShapeDtypeStruct((_NC, n, 128), jnp.float32),
        mesh=mesh,
        scratch_types=[
            pltpu.VMEM((_CH,), jnp.int32),
            pltpu.VMEM((_CH,), jnp.int32),
            pltpu.VMEM((_CH, 128), jnp.float32),
            pltpu.VMEM((_CH, 128), jnp.float32),
            pltpu.VMEM_SHARED((n, 128), jnp.float32),
            pltpu.SemaphoreType.DMA,
        ],
    )
    return f(slin, filt_l, row2d, col2d, zeros_ns)


def _update_body(s_ref, agg_ref, deg_ref, lo_ref, ln_ref, s_out, slin_out, *,
                 has_next):
    agg = agg_ref[0] + agg_ref[1]
    deg = jnp.maximum(deg_ref[0, :, 0:1] + deg_ref[1, :, 0:1], 1.0)
    u = _ssp(agg / deg)
    s_new = s_ref[...] + jnp.dot(u, lo_ref[...],
                                 preferred_element_type=jnp.float32)
    s_out[...] = s_new
    if has_next:
        slin_out[...] = jnp.dot(s_new, ln_ref[...],
                                preferred_element_type=jnp.float32)
    else:
        slin_out[...] = s_new


def _update(s, aggp, degp, lin_out_l, lin_in_next, n, blk, has_next):
    grid = n // blk
    return pl.pallas_call(
        functools.partial(_update_body, has_next=has_next),
        grid=(grid,),
        in_specs=[
            pl.BlockSpec((blk, 128), lambda i: (i, 0)),
            pl.BlockSpec((_NC, blk, 128), lambda i: (0, i, 0)),
            pl.BlockSpec((_NC, blk, 128), lambda i: (0, i, 0)),
            pl.BlockSpec((128, 128), lambda i: (0, 0)),
            pl.BlockSpec((128, 128), lambda i: (0, 0)),
        ],
        out_specs=[
            pl.BlockSpec((blk, 128), lambda i: (i, 0)),
            pl.BlockSpec((blk, 128), lambda i: (i, 0)),
        ],
        out_shape=[
            jax.ShapeDtypeStruct((n, 128), jnp.float32),
            jax.ShapeDtypeStruct((n, 128), jnp.float32),
        ],
    )(s, aggp, degp, lin_out_l, lin_in_next)


def _final_body(s_ref, batch_ref, lig_ref, ch_ref, g_ref, be_ref, pw_ref,
                aw1_ref, ab1_ref, aw2_ref, ab2_ref, pw1_ref, pb1_ref, pw2_ref,
                pb2_ref, aff_ref, prop_ref, *, n, n_graphs, n_chains, n_aff,
                n_prop, prop_classes):
    s = s_ref[...]
    m = jnp.mean(s, axis=1, keepdims=True)
    d = s - m
    var = jnp.mean(d * d, axis=1, keepdims=True)
    sn = g_ref[...] * d * lax.rsqrt(var + 1e-5) + be_ref[...]
    sp = jnp.dot(sn, pw_ref[...], preferred_element_type=jnp.float32)

    bt = batch_ref[...]
    gids = lax.broadcasted_iota(jnp.int32, (n_graphs, 1), 0)
    oh_g = (bt == gids).astype(jnp.float32)
    gsum = jnp.dot(oh_g, sp, preferred_element_type=jnp.float32)
    gcnt = jnp.maximum(jnp.sum(oh_g, axis=1, keepdims=True), 1.0)
    g = gsum / gcnt

    for i in range(n_aff):
        h = _silu(jnp.dot(g, aw1_ref[i], preferred_element_type=jnp.float32)
                  + ab1_ref[i])
        val = jnp.sum(h * aw2_ref[i], axis=1, keepdims=True) + ab2_ref[i, 0, 0]
        aff_ref[i] = jnp.broadcast_to(val, (n_graphs, 128))

    w = (lig_ref[...] != 0).astype(jnp.float32)
    cidx = ch_ref[...] - 1
    cids = lax.broadcasted_iota(jnp.int32, (n_chains, 1), 0)
    oh_c = (cidx == cids).astype(jnp.float32) * w
    csum = jnp.dot(oh_c, sp, preferred_element_type=jnp.float32)
    ccnt = jnp.maximum(jnp.sum(oh_c, axis=1, keepdims=True), 1.0)
    cg = csum / ccnt

    for i in range(n_prop):
        h = _silu(jnp.dot(cg, pw1_ref[i], preferred_element_type=jnp.float32)
                  + pb1_ref[i])
        for c in range(prop_classes):
            val = (jnp.sum(h * pw2_ref[i, c][None, :], axis=1, keepdims=True)
                   + pb2_ref[i, 0, c])
            prop_ref[i * prop_classes + c] = jnp.broadcast_to(
                val, (n_chains, 128))


def _final(s, batch_r, lig_r, ch_r, ln_gamma, ln_beta, post_w, aff_w1, aff_b1,
           aff_w2, aff_b2, prop_w1, prop_b1, prop_w2, prop_b2, n, n_graphs,
           n_chains):
    n_aff = aff_w1.shape[0]
    n_prop = prop_w1.shape[0]
    prop_classes = prop_w2.shape[2]
    body = functools.partial(
        _final_body, n=n, n_graphs=n_graphs, n_chains=n_chains, n_aff=n_aff,
        n_prop=n_prop, prop_classes=prop_classes)
    return pl.pallas_call(
        body,
        out_shape=[
            jax.ShapeDtypeStruct((n_aff, n_graphs, 128), jnp.float32),
            jax.ShapeDtypeStruct((n_prop * prop_classes, n_chains, 128),
                                 jnp.float32),
        ],
    )(s, batch_r, lig_r, ch_r, ln_gamma.reshape(1, 128),
      ln_beta.reshape(1, 128), post_w, aff_w1,
      aff_b1.reshape(n_aff, 1, 128), aff_w2.reshape(n_aff, 1, 128),
      aff_b2.reshape(n_aff, 1, 1), prop_w1, prop_b1.reshape(n_prop, 1, 128),
      jnp.transpose(prop_w2, (0, 2, 1)),
      prop_b2.reshape(n_prop, 1, prop_classes))


def kernel(x, pos, batch, channel_weights, edge_index, edge_weights, lig_flag,
           chains, emb, gnn_w1, gnn_b1, gnn_w2, gnn_b2, gnn_lin_in,
           gnn_lin_out, ln_gamma, ln_beta, post_w, aff_w1, aff_b1, aff_w2,
           aff_b2, prop_w1, prop_b1, prop_w2, prop_b2):
    n = x.shape[0]
    e = edge_index.shape[1]
    depth = gnn_w1.shape[0]
    num_radial = gnn_w1.shape[1]
    n_graphs = 16
    n_chains = 8
    blk_n = _pick_blk(n, (2000, 1000, 500, 250, 200, 8))
    blk_e = _pick_blk(e, (2000, 2048, 1024, 512, 256, 128))

    x2d = x.astype(jnp.int32).reshape(n, 1)
    row2d = edge_index[0].astype(jnp.int32).reshape(e // _CH, _CH)
    col2d = edge_index[1].astype(jnp.int32).reshape(e // _CH, _CH)
    ew2d = edge_weights.reshape(e, 1)
    emb_pad = jnp.zeros((128, 128), jnp.float32).at[:emb.shape[0]].set(emb)
    zeros_ns = jnp.zeros((n, 128), jnp.float32)

    s, slin = _embed(x2d, emb_pad, gnn_lin_in[0], n, blk_n)
    filt = [_filters1(ew2d, gnn_w1[l], gnn_b1[l], gnn_w2[l], gnn_b2[l], e,
                      num_radial, blk_e) for l in range(depth)]
    degp = _degree(row2d, zeros_ns, n, e)

    for l in range(depth):
        aggp = _cfconv(slin, filt[l], row2d, col2d, zeros_ns, n, e)
        has_next = l + 1 < depth
        nxt = gnn_lin_in[l + 1] if has_next else gnn_lin_in[0]
        s, slin = _update(s, aggp, degp, gnn_lin_out[l], nxt, n, blk_n,
                          has_next)

    aff_pad, prop_pad = _final(
        s, batch.astype(jnp.int32).reshape(1, n),
        lig_flag.astype(jnp.int32).reshape(1, n),
        chains.astype(jnp.int32).reshape(1, n), ln_gamma, ln_beta, post_w,
        aff_w1, aff_b1, aff_w2, aff_b2, prop_w1, prop_b1, prop_w2, prop_b2,
        n, n_graphs, n_chains)

    n_aff = aff_w1.shape[0]
    n_prop = prop_w1.shape[0]
    prop_classes = prop_w2.shape[2]
    aff_out = aff_pad[:, :, :1]
    prop_out = jnp.transpose(
        prop_pad[:, :, 0].reshape(n_prop, prop_classes, n_chains), (0, 2, 1))
    return aff_out, prop_out


# speedup vs baseline: 3.8752x; 1.2170x over previous
import functools

import jax
import jax.numpy as jnp
from jax import lax
from jax.experimental import pallas as pl
from jax.experimental.pallas import tpu as pltpu
from jax.experimental.pallas import tpu_sc as plsc

_R_CUTOFF = 4.5
_GAMMA = 10.0
_LN2 = 0.6931471805599453

_NC = 2
_NS = 16
_CH = 128


def _pick_blk(total, cands):
    return next(c for c in cands if total % c == 0)


def _ssp(v):
    return jnp.maximum(v, 0.0) + jnp.log(1.0 + jnp.exp(-jnp.abs(v))) - _LN2


def _silu(v):
    return v / (1.0 + jnp.exp(-v))


def _embed_body(x_ref, emb_ref, lin_ref, s_ref, slin_ref):
    xv = x_ref[...]
    ids = lax.broadcasted_iota(jnp.int32, (1, 128), 1)
    onehot = (xv == ids).astype(jnp.float32)
    s = jnp.dot(onehot, emb_ref[...], preferred_element_type=jnp.float32)
    s_ref[...] = s
    slin_ref[...] = jnp.dot(s, lin_ref[...], preferred_element_type=jnp.float32)


def _embed(x2d, emb_pad, lin_in0, n, blk):
    grid = n // blk
    return pl.pallas_call(
        _embed_body,
        grid=(grid,),
        in_specs=[
            pl.BlockSpec((blk, 1), lambda i: (i, 0)),
            pl.BlockSpec((128, 128), lambda i: (0, 0)),
            pl.BlockSpec((128, 128), lambda i: (0, 0)),
        ],
        out_specs=[
            pl.BlockSpec((blk, 128), lambda i: (i, 0)),
            pl.BlockSpec((blk, 128), lambda i: (i, 0)),
        ],
        out_shape=[
            jax.ShapeDtypeStruct((n, 128), jnp.float32),
            jax.ShapeDtypeStruct((n, 128), jnp.float32),
        ],
    )(x2d, emb_pad, lin_in0)


def _filt_body(ew_ref, w1_ref, b1_ref, w2_ref, b2_ref, out_ref, *, num_radial):
    ew = ew_ref[...].reshape(ew_ref.shape[0], 1)
    mu = lax.broadcasted_iota(jnp.int32, (1, num_radial), 1).astype(
        jnp.float32) * (_R_CUTOFF / (num_radial - 1))
    rbf = jnp.exp(-_GAMMA * (ew - mu) ** 2)
    h = _ssp(jnp.dot(rbf, w1_ref[...], preferred_element_type=jnp.float32)
             + b1_ref[...])
    out_ref[...] = (jnp.dot(h, w2_ref[...], preferred_element_type=jnp.float32)
                    + b2_ref[...])


def _filters1(ew1d, w1l, b1l, w2l, b2l, e, num_radial, blk):
    return pl.pallas_call(
        functools.partial(_filt_body, num_radial=num_radial),
        grid=(e // blk,),
        in_specs=[
            pl.BlockSpec((blk,), lambda i: (i,)),
            pl.BlockSpec((num_radial, 128), lambda i: (0, 0)),
            pl.BlockSpec((1, 128), lambda i: (0, 0)),
            pl.BlockSpec((128, 128), lambda i: (0, 0)),
            pl.BlockSpec((1, 128), lambda i: (0, 0)),
        ],
        out_specs=pl.BlockSpec((blk, 128), lambda i: (i, 0)),
        out_shape=jax.ShapeDtypeStruct((e, 128), jnp.float32),
    )(ew1d, w1l, b1l.reshape(1, 128), w2l, b2l.reshape(1, 128))


def _tile_copy(src, dst, sid, n):
    npt = (n // _NS) // 8 * 8
    rem = n - _NS * npt
    off = pl.multiple_of(sid * npt, 8)
    pltpu.sync_copy(src.at[pl.ds(off, npt)], dst.at[pl.ds(off, npt)])
    if rem:
        @pl.when(sid == _NS - 1)
        def _():
            pltpu.sync_copy(src.at[pl.ds(_NS * npt, rem)],
                            dst.at[pl.ds(_NS * npt, rem)])


def _deg_body(row2d, zeros_ns, out, rowv, onesb, degsh, *, n, rows_per_core,
              rows_per_tile):
    cid = lax.axis_index("c")
    sid = lax.axis_index("s")
    _tile_copy(zeros_ns, degsh, sid, n)

    def fill_row(r, _):
        for c8 in range(8):
            onesb[r, pl.ds(c8 * 16, 16)] = jnp.full((16,), 1.0, jnp.float32)
        return 0

    lax.fori_loop(0, _CH, fill_row, 0)
    plsc.subcore_barrier()

    base = cid * rows_per_core + sid * rows_per_tile

    def do_chunk(ck):
        pltpu.sync_copy(row2d.at[ck], rowv)
        pltpu.sync_copy(onesb, degsh.at[rowv], add=True)

    def chunk_loop(i, _):
        do_chunk(base + i)
        return 0

    lax.fori_loop(0, rows_per_tile, chunk_loop, 0)
    nextra = rows_per_core - _NS * rows_per_tile

    @pl.when(sid < nextra)
    def _():
        do_chunk(cid * rows_per_core + _NS * rows_per_tile + sid)

    plsc.subcore_barrier()
    _tile_copy(degsh, out.at[cid], sid, n)


def _degree(row2d, zeros_ns, n, e):
    nrows = e // _CH
    rows_per_core = nrows // _NC
    rows_per_tile = rows_per_core // _NS
    mesh = plsc.VectorSubcoreMesh(core_axis_name="c", subcore_axis_name="s",
                                  num_cores=_NC, num_subcores=_NS)
    body = functools.partial(_deg_body, n=n, rows_per_core=rows_per_core,
                             rows_per_tile=rows_per_tile)
    f = pl.kernel(
        body,
        out_type=jax.ShapeDtypeStruct((_NC, n, 128), jnp.float32),
        mesh=mesh,
        scratch_types=[
            pltpu.VMEM((_CH,), jnp.int32),
            pltpu.VMEM((_CH, 128), jnp.float32),
            pltpu.VMEM_SHARED((n, 128), jnp.float32),
        ],
    )
    return f(row2d, zeros_ns)


def _cfconv_body(slin, filt, row2d, col2d, zeros_ns, out, colv, rowv, xr, fc,
                 aggsh, sem, *, n, rows_per_core, rows_per_tile):
    cid = lax.axis_index("c")
    sid = lax.axis_index("s")
    _tile_copy(zeros_ns, aggsh, sid, n)
    plsc.subcore_barrier()

    base = cid * rows_per_core + sid * rows_per_tile

    def do_chunk(ck):
        pltpu.sync_copy(col2d.at[ck], colv)
        pltpu.sync_copy(row2d.at[ck], rowv)
        gather = pltpu.async_copy(slin.at[colv], xr, sem)
        off = pl.multiple_of(ck * _CH, _CH)
        pltpu.sync_copy(filt.at[pl.ds(off, _CH)], fc)
        gather.wait()

        def mul_row(r, _):
            for c8 in range(8):
                sl = pl.ds(c8 * 16, 16)
                xr[r, sl] = xr[r, sl] * fc[r, sl]
            return 0

        lax.fori_loop(0, _CH, mul_row, 0)
        pltpu.sync_copy(xr, aggsh.at[rowv], add=True)

    def chunk_loop(i, _):
        do_chunk(base + i)
        return 0

    lax.fori_loop(0, rows_per_tile, chunk_loop, 0)
    nextra = rows_per_core - _NS * rows_per_tile

    @pl.when(sid < nextra)
    def _():
        do_chunk(cid * rows_per_core + _NS * rows_per_tile + sid)

    plsc.subcore_barrier()
    _tile_copy(aggsh, out.at[cid], sid, n)


def _cfconv(slin, filt_l, row2d, col2d, zeros_ns, n, e):
    nrows = e // _CH
    rows_per_core = nrows // _NC
    rows_per_tile = rows_per_core // _NS
    mesh = plsc.VectorSubcoreMesh(core_axis_name="c", subcore_axis_name="s",
                                  num_cores=_NC, num_subcores=_NS)
    body = functools.partial(_cfconv_body, n=n, rows_per_core=rows_per_core,
                             rows_per_tile=rows_per_tile)
    f = pl.kernel(
        body,
        out_type=jax.ShapeDtypeStruct((_NC, n, 128), jnp.float32),
        mesh=mesh,
        scratch_types=[
            pltpu.VMEM((_CH,), jnp.int32),
            pltpu.VMEM((_CH,), jnp.int32),
            pltpu.VMEM((_CH, 128), jnp.float32),
            pltpu.VMEM((_CH, 128), jnp.float32),
            pltpu.VMEM_SHARED((n, 128), jnp.float32),
            pltpu.SemaphoreType.DMA,
        ],
    )
    return f(slin, filt_l, row2d, col2d, zeros_ns)


def _update_body(s_ref, agg_ref, deg_ref, lo_ref, ln_ref, s_out, slin_out, *,
                 has_next):
    agg = agg_ref[0] + agg_ref[1]
    deg = jnp.maximum(deg_ref[0, :, 0:1] + deg_ref[1, :, 0:1], 1.0)
    u = _ssp(agg / deg)
    s_new = s_ref[...] + jnp.dot(u, lo_ref[...],
                                 preferred_element_type=jnp.float32)
    s_out[...] = s_new
    if has_next:
        slin_out[...] = jnp.dot(s_new, ln_ref[...],
                                preferred_element_type=jnp.float32)
    else:
        slin_out[...] = s_new


def _update(s, aggp, degp, lin_out_l, lin_in_next, n, blk, has_next):
    grid = n // blk
    return pl.pallas_call(
        functools.partial(_update_body, has_next=has_next),
        grid=(grid,),
        in_specs=[
            pl.BlockSpec((blk, 128), lambda i: (i, 0)),
            pl.BlockSpec((_NC, blk, 128), lambda i: (0, i, 0)),
            pl.BlockSpec((_NC, blk, 128), lambda i: (0, i, 0)),
            pl.BlockSpec((128, 128), lambda i: (0, 0)),
            pl.BlockSpec((128, 128), lambda i: (0, 0)),
        ],
        out_specs=[
            pl.BlockSpec((blk, 128), lambda i: (i, 0)),
            pl.BlockSpec((blk, 128), lambda i: (i, 0)),
        ],
        out_shape=[
            jax.ShapeDtypeStruct((n, 128), jnp.float32),
            jax.ShapeDtypeStruct((n, 128), jnp.float32),
        ],
    )(s, aggp, degp, lin_out_l, lin_in_next)


def _final_body(s_ref, batch_ref, lig_ref, ch_ref, g_ref, be_ref, pw_ref,
                aw1_ref, ab1_ref, aw2_ref, ab2_ref, pw1_ref, pb1_ref, pw2_ref,
                pb2_ref, aff_ref, prop_ref, *, n, n_graphs, n_chains, n_aff,
                n_prop, prop_classes):
    s = s_ref[...]
    m = jnp.mean(s, axis=1, keepdims=True)
    d = s - m
    var = jnp.mean(d * d, axis=1, keepdims=True)
    sn = g_ref[...] * d * lax.rsqrt(var + 1e-5) + be_ref[...]
    sp = jnp.dot(sn, pw_ref[...], preferred_element_type=jnp.float32)

    bt = batch_ref[...]
    gids = lax.broadcasted_iota(jnp.int32, (n_graphs, 1), 0)
    oh_g = (bt == gids).astype(jnp.float32)
    gsum = jnp.dot(oh_g, sp, preferred_element_type=jnp.float32)
    gcnt = jnp.maximum(jnp.sum(oh_g, axis=1, keepdims=True), 1.0)
    g = gsum / gcnt

    for i in range(n_aff):
        h = _silu(jnp.dot(g, aw1_ref[i], preferred_element_type=jnp.float32)
                  + ab1_ref[i])
        val = jnp.sum(h * aw2_ref[i], axis=1, keepdims=True) + ab2_ref[i, 0, 0]
        aff_ref[i] = jnp.broadcast_to(val, (n_graphs, 128))

    w = (lig_ref[...] != 0).astype(jnp.float32)
    cidx = ch_ref[...] - 1
    cids = lax.broadcasted_iota(jnp.int32, (n_chains, 1), 0)
    oh_c = (cidx == cids).astype(jnp.float32) * w
    csum = jnp.dot(oh_c, sp, preferred_element_type=jnp.float32)
    ccnt = jnp.maximum(jnp.sum(oh_c, axis=1, keepdims=True), 1.0)
    cg = csum / ccnt

    for i in range(n_prop):
        h = _silu(jnp.dot(cg, pw1_ref[i], preferred_element_type=jnp.float32)
                  + pb1_ref[i])
        for c in range(prop_classes):
            val = (jnp.sum(h * pw2_ref[i, c][None, :], axis=1, keepdims=True)
                   + pb2_ref[i, 0, c])
            prop_ref[i * prop_classes + c] = jnp.broadcast_to(
                val, (n_chains, 128))


def _final(s, batch_r, lig_r, ch_r, ln_gamma, ln_beta, post_w, aff_w1, aff_b1,
           aff_w2, aff_b2, prop_w1, prop_b1, prop_w2, prop_b2, n, n_graphs,
           n_chains):
    n_aff = aff_w1.shape[0]
    n_prop = prop_w1.shape[0]
    prop_classes = prop_w2.shape[2]
    body = functools.partial(
        _final_body, n=n, n_graphs=n_graphs, n_chains=n_chains, n_aff=n_aff,
        n_prop=n_prop, prop_classes=prop_classes)
    return pl.pallas_call(
        body,
        out_shape=[
            jax.ShapeDtypeStruct((n_aff, n_graphs, 128), jnp.float32),
            jax.ShapeDtypeStruct((n_prop * prop_classes, n_chains, 128),
                                 jnp.float32),
        ],
    )(s, batch_r, lig_r, ch_r, ln_gamma.reshape(1, 128),
      ln_beta.reshape(1, 128), post_w, aff_w1,
      aff_b1.reshape(n_aff, 1, 128), aff_w2.reshape(n_aff, 1, 128),
      aff_b2.reshape(n_aff, 1, 1), prop_w1, prop_b1.reshape(n_prop, 1, 128),
      jnp.transpose(prop_w2, (0, 2, 1)),
      prop_b2.reshape(n_prop, 1, prop_classes))


def kernel(x, pos, batch, channel_weights, edge_index, edge_weights, lig_flag,
           chains, emb, gnn_w1, gnn_b1, gnn_w2, gnn_b2, gnn_lin_in,
           gnn_lin_out, ln_gamma, ln_beta, post_w, aff_w1, aff_b1, aff_w2,
           aff_b2, prop_w1, prop_b1, prop_w2, prop_b2):
    n = x.shape[0]
    e = edge_index.shape[1]
    depth = gnn_w1.shape[0]
    num_radial = gnn_w1.shape[1]
    n_graphs = 16
    n_chains = 8
    blk_n = _pick_blk(n, (2000, 1000, 500, 250, 200, 8))
    blk_e = _pick_blk(e, (2000, 2048, 1024, 512, 256, 128))

    x2d = x.astype(jnp.int32).reshape(n, 1)
    row2d = edge_index[0].astype(jnp.int32).reshape(e // _CH, _CH)
    col2d = edge_index[1].astype(jnp.int32).reshape(e // _CH, _CH)
    e_pad = (e + 2047) // 2048 * 2048
    ew1d = jnp.pad(edge_weights.reshape(e), (0, e_pad - e))
    emb_pad = jnp.zeros((128, 128), jnp.float32).at[:emb.shape[0]].set(emb)
    zeros_ns = jnp.zeros((n, 128), jnp.float32)

    s, slin = _embed(x2d, emb_pad, gnn_lin_in[0], n, blk_n)
    filt = [_filters1(ew1d, gnn_w1[l], gnn_b1[l], gnn_w2[l], gnn_b2[l],
                      e_pad, num_radial, 2048) for l in range(depth)]
    degp = _degree(row2d, zeros_ns, n, e)
    slin, _ = lax.optimization_barrier((slin, degp))

    for l in range(depth):
        aggp = _cfconv(slin, filt[l], row2d, col2d, zeros_ns, n, e)
        has_next = l + 1 < depth
        nxt = gnn_lin_in[l + 1] if has_next else gnn_lin_in[0]
        s, slin = _update(s, aggp, degp, gnn_lin_out[l], nxt, n, blk_n,
                          has_next)

    aff_pad, prop_pad = _final(
        s, batch.astype(jnp.int32).reshape(1, n),
        lig_flag.astype(jnp.int32).reshape(1, n),
        chains.astype(jnp.int32).reshape(1, n), ln_gamma, ln_beta, post_w,
        aff_w1, aff_b1, aff_w2, aff_b2, prop_w1, prop_b1, prop_w2, prop_b2,
        n, n_graphs, n_chains)

    n_aff = aff_w1.shape[0]
    n_prop = prop_w1.shape[0]
    prop_classes = prop_w2.shape[2]
    aff_out = aff_pad[:, :, :1]
    prop_out = jnp.transpose(
        prop_pad[:, :, 0].reshape(n_prop, prop_classes, n_chains), (0, 2, 1))
    return aff_out, prop_out
